# Initial kernel scaffold; baseline (speedup 1.0000x reference)
import functools

import jax
import jax.numpy as jnp
from jax import lax
from jax.experimental import pallas as pl
from jax.experimental.pallas import tpu as pltpu
import jax.experimental.pallas.tpu_sc as plsc

_NC = 2
_NS = 16
_NW = _NC * _NS
_CHUNK = 80


@functools.lru_cache(maxsize=None)
def _make_agg(N, E, D, with_count):
    e_per_w = E // _NW
    n_chunks = e_per_w // _CHUNK
    rows_per_tile = N // _NS
    mesh = plsc.VectorSubcoreMesh(core_axis_name="c", subcore_axis_name="s")

    out_type = [jax.ShapeDtypeStruct((_NC, N, D), jnp.float32)]
    scratch = [
        pltpu.VMEM((_CHUNK,), jnp.int32),
        pltpu.VMEM((_CHUNK,), jnp.int32),
        pltpu.VMEM((_CHUNK, D), jnp.float32),
        pltpu.VMEM_SHARED((N, D), jnp.float32),
        pltpu.SemaphoreType.DMA,
    ]
    if with_count:
        out_type.append(jax.ShapeDtypeStruct((_NC, N, 16), jnp.float32))
        scratch += [
            pltpu.VMEM((_CHUNK, 16), jnp.float32),
            pltpu.VMEM_SHARED((N, 16), jnp.float32),
        ]

    def body(y_hbm, src_hbm, dst_hbm, zrow_hbm, zcnt_hbm, ones_hbm,
             out_hbm, cnt_hbm, sidx, didx, rows, acc, sem,
             ones_v=None, cacc=None):
        c = lax.axis_index("c")
        s = lax.axis_index("s")
        wid = c * _NS + s
        r0 = s * rows_per_tile
        pltpu.sync_copy(zrow_hbm.at[pl.ds(r0, rows_per_tile)],
                        acc.at[pl.ds(r0, rows_per_tile)])
        if with_count:
            pltpu.sync_copy(zcnt_hbm.at[pl.ds(r0, rows_per_tile)],
                            cacc.at[pl.ds(r0, rows_per_tile)])
            pltpu.sync_copy(ones_hbm, ones_v)
        plsc.subcore_barrier()
        base_e = wid * e_per_w

        def step(i, carry):
            eb = base_e + i * _CHUNK
            pltpu.sync_copy(src_hbm.at[pl.ds(eb, _CHUNK)], sidx)
            pltpu.async_copy(y_hbm.at[sidx], rows, sem).wait()
            pltpu.sync_copy(dst_hbm.at[pl.ds(eb, _CHUNK)], didx)
            pltpu.sync_copy(rows, acc.at[didx], add=True)
            if with_count:
                pltpu.sync_copy(ones_v, cacc.at[didx], add=True)
            return carry

        lax.fori_loop(0, n_chunks, step, 0)
        plsc.subcore_barrier()
        pltpu.sync_copy(acc.at[pl.ds(r0, rows_per_tile)],
                        out_hbm.at[c, pl.ds(r0, rows_per_tile)])
        if with_count:
            pltpu.sync_copy(cacc.at[pl.ds(r0, rows_per_tile)],
                            cnt_hbm.at[c, pl.ds(r0, rows_per_tile)])

    if not with_count:
        def body_nc(y_hbm, src_hbm, dst_hbm, zrow_hbm, out_hbm,
                    sidx, didx, rows, acc, sem):
            body(y_hbm, src_hbm, dst_hbm, zrow_hbm, None, None,
                 out_hbm, None, sidx, didx, rows, acc, sem)
        fn = body_nc
    else:
        fn = body

    return pl.kernel(fn, out_type=tuple(out_type), mesh=mesh,
                     scratch_types=tuple(scratch))


@functools.lru_cache(maxsize=None)
def _make_combine(N, D, H, BLOCK):
    grid = N // BLOCK

    def body(p0_ref, p1_ref, c0_ref, c1_ref, x_ref, b_ref, wl_ref, wr_ref,
             o_ref):
        cnt = c0_ref[:, 0:1] + c1_ref[:, 0:1]
        inv = 1.0 / jnp.maximum(cnt, 1.0)
        mean = (p0_ref[...] + p1_ref[...]) * inv
        o = (jnp.dot(mean, wl_ref[...], preferred_element_type=jnp.float32)
             + b_ref[...]
             + jnp.dot(x_ref[...], wr_ref[...],
                       preferred_element_type=jnp.float32))
        o_ref[...] = jnp.maximum(o, 0.0)

    row_spec = pl.BlockSpec((BLOCK, D), lambda i: (i, 0))
    cnt_spec = pl.BlockSpec((BLOCK, 16), lambda i: (i, 0))
    full = lambda shape: pl.BlockSpec(shape, lambda i: (0, 0))
    return pl.pallas_call(
        body,
        grid=(grid,),
        in_specs=[row_spec, row_spec, cnt_spec, cnt_spec, row_spec,
                  full((1, H)), full((D, H)), full((D, H))],
        out_specs=pl.BlockSpec((BLOCK, H), lambda i: (i, 0)),
        out_shape=jax.ShapeDtypeStruct((N, H), jnp.float32),
    )


def kernel(x, edge_index, W_l1, b_l1, W_r1, W_l2, b_l2, W_r2):
    N, D = x.shape
    H = W_l1.shape[0]
    O = W_l2.shape[0]
    E = edge_index.shape[1]
    src = edge_index[0]
    dst = edge_index[1]
    zrow = jnp.zeros((N, D), jnp.float32)
    zcnt = jnp.zeros((N, 16), jnp.float32)
    ones = jnp.ones((_CHUNK, 16), jnp.float32)

    agg1 = _make_agg(N, E, D, True)
    agg2 = _make_agg(N, E, H, False)
    combine1 = _make_combine(N, D, H, 1000)
    combine2 = _make_combine(N, H, O, 1000)

    p, cnt = agg1(x, src, dst, zrow, zcnt, ones)
    h = combine1(p[0], p[1], cnt[0], cnt[1], x, b_l1.reshape(1, H),
                 W_l1.T, W_r1.T)
    (q,) = agg2(h, src, dst, zrow)
    out = combine2(q[0], q[1], cnt[0], cnt[1], h, b_l2.reshape(1, O),
                   W_l2.T, W_r2.T)
    return out


# speedup vs baseline: 1.8842x; 1.8842x over previous
import functools

import jax
import jax.numpy as jnp
from jax import lax
from jax.experimental import pallas as pl
from jax.experimental.pallas import tpu as pltpu
import jax.experimental.pallas.tpu_sc as plsc

_NC = 2
_NS = 16
_NW = _NC * _NS
_CHUNK = 80


@functools.lru_cache(maxsize=None)
def _make_agg(N, E, D, with_count):
    e_per_w = E // _NW
    n_chunks = e_per_w // _CHUNK
    rows_per_tile = -(-N // (_NS * 128)) * 128
    N_pad = rows_per_tile * _NS
    CR = N_pad // 8
    mesh = plsc.VectorSubcoreMesh(core_axis_name="c", subcore_axis_name="s")

    out_type = [jax.ShapeDtypeStruct((_NC * N_pad, D), jnp.float32)]
    scratch = [
        pltpu.VMEM((_CHUNK,), jnp.int32),
        pltpu.VMEM((_CHUNK,), jnp.int32),
        pltpu.VMEM((_CHUNK, D), jnp.float32),
        pltpu.VMEM_SHARED((N_pad, D), jnp.float32),
        pltpu.SemaphoreType.DMA,
    ]
    if with_count:
        out_type.append(jax.ShapeDtypeStruct((_NC * CR, 128), jnp.float32))
        scratch += [
            pltpu.VMEM((_CHUNK,), jnp.int32),
            pltpu.VMEM((_CHUNK,), jnp.int32),
            pltpu.VMEM((_CHUNK, 128), jnp.float32),
            pltpu.VMEM_SHARED((CR, 128), jnp.float32),
        ]

    def body(y_hbm, src_hbm, dst_hbm, zrow_hbm, pat_hbm, out_hbm, cnt_hbm,
             sidx, didx, rows, acc, sem, pidx=None, crow=None, prow=None,
             cacc=None):
        c = lax.axis_index("c")
        s = lax.axis_index("s")
        wid = c * _NS + s
        r0 = s * rows_per_tile
        pltpu.sync_copy(zrow_hbm.at[pl.ds(r0, rows_per_tile)],
                        acc.at[pl.ds(r0, rows_per_tile)])
        if with_count:
            cz = CR // _NS
            pltpu.sync_copy(zrow_hbm.at[pl.ds(s * cz, cz)],
                            cacc.at[pl.ds(s * cz, cz)])
        plsc.subcore_barrier()
        base_e = wid * e_per_w

        def step(i, carry):
            eb = base_e + i * _CHUNK
            pltpu.sync_copy(src_hbm.at[pl.ds(eb, _CHUNK)], sidx)
            pltpu.async_copy(y_hbm.at[sidx], rows, sem).wait()
            pltpu.sync_copy(dst_hbm.at[pl.ds(eb, _CHUNK)], didx)
            pltpu.sync_copy(rows, acc.at[didx], add=True)
            if with_count:
                for k in range(_CHUNK // 16):
                    n16 = didx[pl.ds(k * 16, 16)]
                    pidx[pl.ds(k * 16, 16)] = lax.bitwise_and(n16, 7)
                    crow[pl.ds(k * 16, 16)] = lax.shift_right_logical(n16, 3)
                pltpu.async_copy(pat_hbm.at[pidx], prow, sem).wait()
                pltpu.sync_copy(prow, cacc.at[crow], add=True)
            return carry

        lax.fori_loop(0, n_chunks, step, 0)
        plsc.subcore_barrier()
        pltpu.sync_copy(acc.at[pl.ds(r0, rows_per_tile)],
                        out_hbm.at[pl.ds(c * N_pad + r0, rows_per_tile)])
        if with_count:
            cz = CR // _NS
            pltpu.sync_copy(cacc.at[pl.ds(s * cz, cz)],
                            cnt_hbm.at[pl.ds(c * CR + s * cz, cz)])

    if with_count:
        fn = body
        ot = tuple(out_type)
    else:
        def fn(y_hbm, src_hbm, dst_hbm, zrow_hbm, out_hbm,
               sidx, didx, rows, acc, sem):
            body(y_hbm, src_hbm, dst_hbm, zrow_hbm, None, out_hbm, None,
                 sidx, didx, rows, acc, sem)
        ot = out_type[0]

    return pl.kernel(fn, out_type=ot, mesh=mesh,
                     scratch_types=tuple(scratch))


@functools.lru_cache(maxsize=None)
def _make_combine(N, D, H, BLOCK):
    grid = N // BLOCK

    def body(p0_ref, p1_ref, c0_ref, c1_ref, x_ref, b_ref, wl_ref, wr_ref,
             o_ref):
        cnt = c0_ref[...] + c1_ref[...]
        inv = 1.0 / jnp.maximum(cnt, 1.0)
        mean = (p0_ref[...] + p1_ref[...]) * inv
        o = (jnp.dot(mean, wl_ref[...], preferred_element_type=jnp.float32)
             + b_ref[...]
             + jnp.dot(x_ref[...], wr_ref[...],
                       preferred_element_type=jnp.float32))
        o_ref[...] = jnp.maximum(o, 0.0)

    row_spec = pl.BlockSpec((BLOCK, D), lambda i: (i, 0))
    cnt_spec = pl.BlockSpec((BLOCK, 1), lambda i: (i, 0))
    full = lambda shape: pl.BlockSpec(shape, lambda i: (0, 0))
    return pl.pallas_call(
        body,
        grid=(grid,),
        in_specs=[row_spec, row_spec, cnt_spec, cnt_spec, row_spec,
                  full((1, H)), full((D, H)), full((D, H))],
        out_specs=pl.BlockSpec((BLOCK, H), lambda i: (i, 0)),
        out_shape=jax.ShapeDtypeStruct((N, H), jnp.float32),
    )


def kernel(x, edge_index, W_l1, b_l1, W_r1, W_l2, b_l2, W_r2):
    N, D = x.shape
    H = W_l1.shape[0]
    O = W_l2.shape[0]
    E = edge_index.shape[1]
    src = edge_index[0]
    dst = edge_index[1]
    rows_per_tile = -(-N // (_NS * 128)) * 128
    N_pad = rows_per_tile * _NS
    zrow = jnp.zeros((N_pad, D), jnp.float32)
    pat = (jnp.arange(128)[None, :] // 16 == jnp.arange(8)[:, None]
           ).astype(jnp.float32)

    agg1 = _make_agg(N, E, D, True)
    agg2 = _make_agg(N, E, H, False)
    combine1 = _make_combine(N, D, H, 1000)
    combine2 = _make_combine(N, H, O, 1000)

    p, cnt = agg1(x, src, dst, zrow, pat)
    cnt = cnt.reshape(_NC, N_pad // 8, 8, 16)[:, :, :, 0].reshape(_NC, N_pad)
    c0 = cnt[0, :N, None]
    c1 = cnt[1, :N, None]
    h = combine1(p[:N], p[N_pad:N_pad + N], c0, c1, x, b_l1.reshape(1, H),
                 W_l1.T, W_r1.T)
    q = agg2(h, src, dst, zrow)
    out = combine2(q[:N], q[N_pad:N_pad + N], c0, c1, h, b_l2.reshape(1, O),
                   W_l2.T, W_r2.T)
    return out


# speedup vs baseline: 4.6785x; 2.4831x over previous
import functools

import jax
import jax.numpy as jnp
from jax import lax
from jax.experimental import pallas as pl
from jax.experimental.pallas import tpu as pltpu
import jax.experimental.pallas.tpu_sc as plsc

_NC = 2
_NS = 16
_NW = _NC * _NS
_CHUNK = 80


@functools.lru_cache(maxsize=None)
def _make_agg(N, E, D):
    e_per_w = E // _NW
    n_chunks = e_per_w // _CHUNK
    rows_per_tile = -(-N // (_NS * 128)) * 128
    N_pad = rows_per_tile * _NS
    mesh = plsc.VectorSubcoreMesh(core_axis_name="c", subcore_axis_name="s")

    def body(y_hbm, src_hbm, dst_hbm, zrow_hbm, out_hbm,
             sidx, didx, rows, acc, sem):
        c = lax.axis_index("c")
        s = lax.axis_index("s")
        wid = c * _NS + s
        r0 = s * rows_per_tile
        base_e = wid * e_per_w
        pltpu.sync_copy(zrow_hbm.at[pl.ds(r0, rows_per_tile)],
                        acc.at[pl.ds(r0, rows_per_tile)])
        plsc.subcore_barrier()

        def step(i, carry):
            eb = base_e + i * _CHUNK
            pltpu.sync_copy(src_hbm.at[pl.ds(eb, _CHUNK)], sidx)
            pltpu.async_copy(y_hbm.at[sidx], rows, sem).wait()
            pltpu.sync_copy(dst_hbm.at[pl.ds(eb, _CHUNK)], didx)
            pltpu.sync_copy(rows, acc.at[didx], add=True)
            return carry

        lax.fori_loop(0, n_chunks, step, 0)
        plsc.subcore_barrier()
        pltpu.sync_copy(acc.at[pl.ds(r0, rows_per_tile)],
                        out_hbm.at[pl.ds(c * N_pad + r0, rows_per_tile)])

    return pl.kernel(
        body,
        out_type=jax.ShapeDtypeStruct((_NC * N_pad, D), jnp.float32),
        mesh=mesh,
        scratch_types=(
            pltpu.VMEM((_CHUNK,), jnp.int32),
            pltpu.VMEM((_CHUNK,), jnp.int32),
            pltpu.VMEM((_CHUNK, D), jnp.float32),
            pltpu.VMEM_SHARED((N_pad, D), jnp.float32),
            pltpu.SemaphoreType.DMA,
        ),
    )


@functools.lru_cache(maxsize=None)
def _make_count(N, E):
    e_per_w = E // _NW
    n_chunks = e_per_w // _CHUNK
    rows_per_tile = -(-N // (_NS * 128)) * 128
    N_pad = rows_per_tile * _NS
    mesh = plsc.VectorSubcoreMesh(core_axis_name="c", subcore_axis_name="s")

    def body(dst_hbm, zrow_hbm, ones_hbm, out_hbm, didx, ones_v, acc, sem):
        c = lax.axis_index("c")
        s = lax.axis_index("s")
        wid = c * _NS + s
        r0 = s * rows_per_tile
        base_e = wid * e_per_w
        pltpu.sync_copy(zrow_hbm.at[pl.ds(r0, rows_per_tile)],
                        acc.at[pl.ds(r0, rows_per_tile)])
        pltpu.sync_copy(ones_hbm, ones_v)
        plsc.subcore_barrier()

        def step(i, carry):
            eb = base_e + i * _CHUNK
            pltpu.sync_copy(dst_hbm.at[pl.ds(eb, _CHUNK)], didx)
            pltpu.sync_copy(ones_v, acc.at[didx], add=True)
            return carry

        lax.fori_loop(0, n_chunks, step, 0)
        plsc.subcore_barrier()
        pltpu.sync_copy(acc.at[pl.ds(r0, rows_per_tile)],
                        out_hbm.at[pl.ds(c * N_pad + r0, rows_per_tile)])

    return pl.kernel(
        body,
        out_type=jax.ShapeDtypeStruct((_NC * N_pad, 128), jnp.float32),
        mesh=mesh,
        scratch_types=(
            pltpu.VMEM((_CHUNK,), jnp.int32),
            pltpu.VMEM((_CHUNK, 128), jnp.float32),
            pltpu.VMEM_SHARED((N_pad, 128), jnp.float32),
            pltpu.SemaphoreType.DMA,
        ),
    )


@functools.lru_cache(maxsize=None)
def _make_combine(N, D, H, BLOCK):
    grid = N // BLOCK

    def body(p0_ref, p1_ref, c0_ref, c1_ref, x_ref, b_ref, wl_ref, wr_ref,
             o_ref):
        cnt = c0_ref[...] + c1_ref[...]
        inv = 1.0 / jnp.maximum(cnt, 1.0)
        mean = (p0_ref[...] + p1_ref[...]) * inv
        o = (jnp.dot(mean, wl_ref[...], preferred_element_type=jnp.float32)
             + b_ref[...]
             + jnp.dot(x_ref[...], wr_ref[...],
                       preferred_element_type=jnp.float32))
        o_ref[...] = jnp.maximum(o, 0.0)

    row_spec = pl.BlockSpec((BLOCK, D), lambda i: (i, 0))
    cnt_spec = pl.BlockSpec((BLOCK, 1), lambda i: (i, 0))
    full = lambda shape: pl.BlockSpec(shape, lambda i: (0, 0))
    return pl.pallas_call(
        body,
        grid=(grid,),
        in_specs=[row_spec, row_spec, cnt_spec, cnt_spec, row_spec,
                  full((1, H)), full((D, H)), full((D, H))],
        out_specs=pl.BlockSpec((BLOCK, H), lambda i: (i, 0)),
        out_shape=jax.ShapeDtypeStruct((N, H), jnp.float32),
    )


def kernel(x, edge_index, W_l1, b_l1, W_r1, W_l2, b_l2, W_r2):
    N, D = x.shape
    H = W_l1.shape[0]
    O = W_l2.shape[0]
    E = edge_index.shape[1]
    src = edge_index[0]
    dst = edge_index[1]
    rows_per_tile = -(-N // (_NS * 128)) * 128
    N_pad = rows_per_tile * _NS
    zrow = jnp.zeros((N_pad, D), jnp.float32)

    onesrow = jnp.ones((_CHUNK, 128), jnp.float32)

    agg = _make_agg(N, E, D)
    cntk = _make_count(N, E)
    combine1 = _make_combine(N, D, H, 1000)
    combine2 = _make_combine(N, H, O, 1000)

    cout = cntk(dst, zrow, onesrow)
    c0 = cout[:N, 0:1]
    c1 = cout[N_pad:N_pad + N, 0:1]
    p = agg(x, src, dst, zrow)
    h = combine1(p[:N], p[N_pad:N_pad + N], c0, c1, x, b_l1.reshape(1, H),
                 W_l1.T, W_r1.T)
    q = agg(h, src, dst, zrow)
    out = combine2(q[:N], q[N_pad:N_pad + N], c0, c1, h, b_l2.reshape(1, O),
                   W_l2.T, W_r2.T)
    return out


# speedup vs baseline: 9.9185x; 2.1200x over previous
import functools

import jax
import jax.numpy as jnp
from jax import lax
from jax.experimental import pallas as pl
from jax.experimental.pallas import tpu as pltpu
import jax.experimental.pallas.tpu_sc as plsc

_NC = 2
_NS = 16
_NW = _NC * _NS
_CHUNK = 80


@functools.lru_cache(maxsize=None)
def _make_agg(N, E, D):
    e_per_w = E // _NW
    n_chunks = e_per_w // _CHUNK
    assert n_chunks % 2 == 1
    rows_per_tile = -(-N // (_NS * 128)) * 128
    N_pad = rows_per_tile * _NS
    mesh = plsc.VectorSubcoreMesh(core_axis_name="c", subcore_axis_name="s")

    def body(y_hbm, src_hbm, dst_hbm, zrow_hbm, out_hbm,
             sall, didx2, rows2, acc, semg0, semg1, semd0, semd1):
        c = lax.axis_index("c")
        s = lax.axis_index("s")
        wid = c * _NS + s
        r0 = s * rows_per_tile
        base_e = wid * e_per_w
        semg = (semg0, semg1)
        semd = (semd0, semd1)

        def sidx_of(i):
            return sall.at[pl.ds(i * _CHUNK, _CHUNK)]

        def start(i, b):
            pltpu.async_copy(dst_hbm.at[pl.ds(base_e + i * _CHUNK, _CHUNK)],
                             didx2.at[b], semd[b])
            pltpu.async_copy(y_hbm.at[sidx_of(i)], rows2.at[b], semg[b])

        def finish(i, b):
            pltpu.make_async_copy(
                dst_hbm.at[pl.ds(base_e + i * _CHUNK, _CHUNK)],
                didx2.at[b], semd[b]).wait()
            pltpu.make_async_copy(y_hbm.at[sidx_of(i)], rows2.at[b],
                                  semg[b]).wait()
            pltpu.sync_copy(rows2.at[b], acc.at[didx2.at[b]], add=True)

        pltpu.sync_copy(zrow_hbm.at[pl.ds(r0, rows_per_tile)],
                        acc.at[pl.ds(r0, rows_per_tile)])
        pltpu.sync_copy(src_hbm.at[pl.ds(base_e, e_per_w)], sall)
        plsc.subcore_barrier()

        start(0, 0)

        def step(g, carry):
            i = g * 2
            start(i + 1, 1)
            finish(i, 0)
            start(i + 2, 0)
            finish(i + 1, 1)
            return carry

        lax.fori_loop(0, (n_chunks - 1) // 2, step, 0)
        finish(n_chunks - 1, 0)
        plsc.subcore_barrier()
        pltpu.sync_copy(acc.at[pl.ds(r0, rows_per_tile)],
                        out_hbm.at[pl.ds(c * N_pad + r0, rows_per_tile)])

    return pl.kernel(
        body,
        out_type=jax.ShapeDtypeStruct((_NC * N_pad, D), jnp.float32),
        mesh=mesh,
        scratch_types=(
            pltpu.VMEM((e_per_w,), jnp.int32),
            pltpu.VMEM((2, _CHUNK), jnp.int32),
            pltpu.VMEM((2, _CHUNK, D), jnp.float32),
            pltpu.VMEM_SHARED((N_pad, D), jnp.float32),
            pltpu.SemaphoreType.DMA,
            pltpu.SemaphoreType.DMA,
            pltpu.SemaphoreType.DMA,
            pltpu.SemaphoreType.DMA,
        ),
    )


@functools.lru_cache(maxsize=None)
def _make_count(N, E):
    e_per_w = E // _NW
    n_chunks = e_per_w // _CHUNK
    rows_per_tile = -(-N // (_NS * 128)) * 128
    N_pad = rows_per_tile * _NS
    mesh = plsc.VectorSubcoreMesh(core_axis_name="c", subcore_axis_name="s")

    assert n_chunks % 2 == 1

    def body(dst_hbm, zrow_hbm, ones_hbm, out_hbm, didx2, ones_v, acc,
             semd0, semd1):
        c = lax.axis_index("c")
        s = lax.axis_index("s")
        wid = c * _NS + s
        r0 = s * rows_per_tile
        base_e = wid * e_per_w
        semd = (semd0, semd1)

        def start(i, b):
            pltpu.async_copy(dst_hbm.at[pl.ds(base_e + i * _CHUNK, _CHUNK)],
                             didx2.at[b], semd[b])

        def finish(i, b):
            pltpu.make_async_copy(
                dst_hbm.at[pl.ds(base_e + i * _CHUNK, _CHUNK)],
                didx2.at[b], semd[b]).wait()
            pltpu.sync_copy(ones_v, acc.at[didx2.at[b]], add=True)

        pltpu.sync_copy(zrow_hbm.at[pl.ds(r0, rows_per_tile)],
                        acc.at[pl.ds(r0, rows_per_tile)])
        pltpu.sync_copy(ones_hbm, ones_v)
        plsc.subcore_barrier()

        start(0, 0)

        def step(g, carry):
            i = g * 2
            start(i + 1, 1)
            finish(i, 0)
            start(i + 2, 0)
            finish(i + 1, 1)
            return carry

        lax.fori_loop(0, (n_chunks - 1) // 2, step, 0)
        finish(n_chunks - 1, 0)
        plsc.subcore_barrier()
        pltpu.sync_copy(acc.at[pl.ds(r0, rows_per_tile)],
                        out_hbm.at[pl.ds(c * N_pad + r0, rows_per_tile)])

    return pl.kernel(
        body,
        out_type=jax.ShapeDtypeStruct((_NC * N_pad, 128), jnp.float32),
        mesh=mesh,
        scratch_types=(
            pltpu.VMEM((2, _CHUNK), jnp.int32),
            pltpu.VMEM((_CHUNK, 128), jnp.float32),
            pltpu.VMEM_SHARED((N_pad, 128), jnp.float32),
            pltpu.SemaphoreType.DMA,
            pltpu.SemaphoreType.DMA,
        ),
    )


@functools.lru_cache(maxsize=None)
def _make_combine(N, D, H, BLOCK):
    grid = N // BLOCK

    def body(p0_ref, p1_ref, c0_ref, c1_ref, x_ref, b_ref, wl_ref, wr_ref,
             o_ref):
        cnt = c0_ref[...] + c1_ref[...]
        inv = 1.0 / jnp.maximum(cnt, 1.0)
        mean = (p0_ref[...] + p1_ref[...]) * inv
        o = (jnp.dot(mean, wl_ref[...], preferred_element_type=jnp.float32)
             + b_ref[...]
             + jnp.dot(x_ref[...], wr_ref[...],
                       preferred_element_type=jnp.float32))
        o_ref[...] = jnp.maximum(o, 0.0)

    row_spec = pl.BlockSpec((BLOCK, D), lambda i: (i, 0))
    cnt_spec = pl.BlockSpec((BLOCK, 1), lambda i: (i, 0))
    full = lambda shape: pl.BlockSpec(shape, lambda i: (0, 0))
    return pl.pallas_call(
        body,
        grid=(grid,),
        in_specs=[row_spec, row_spec, cnt_spec, cnt_spec, row_spec,
                  full((1, H)), full((D, H)), full((D, H))],
        out_specs=pl.BlockSpec((BLOCK, H), lambda i: (i, 0)),
        out_shape=jax.ShapeDtypeStruct((N, H), jnp.float32),
    )


def kernel(x, edge_index, W_l1, b_l1, W_r1, W_l2, b_l2, W_r2):
    N, D = x.shape
    H = W_l1.shape[0]
    O = W_l2.shape[0]
    E = edge_index.shape[1]
    src = edge_index[0]
    dst = edge_index[1]
    rows_per_tile = -(-N // (_NS * 128)) * 128
    N_pad = rows_per_tile * _NS
    zrow = jnp.zeros((N_pad, D), jnp.float32)

    onesrow = jnp.ones((_CHUNK, 128), jnp.float32)

    agg = _make_agg(N, E, D)
    cntk = _make_count(N, E)
    combine1 = _make_combine(N, D, H, 1000)
    combine2 = _make_combine(N, H, O, 1000)

    cout = cntk(dst, zrow, onesrow)
    c0 = cout[:N, 0:1]
    c1 = cout[N_pad:N_pad + N, 0:1]
    p = agg(x, src, dst, zrow)
    h = combine1(p[:N], p[N_pad:N_pad + N], c0, c1, x, b_l1.reshape(1, H),
                 W_l1.T, W_r1.T)
    q = agg(h, src, dst, zrow)
    out = combine2(q[:N], q[N_pad:N_pad + N], c0, c1, h, b_l2.reshape(1, O),
                   W_l2.T, W_r2.T)
    return out


# speedup vs baseline: 10.6660x; 1.0754x over previous
import functools

import jax
import jax.numpy as jnp
from jax import lax
from jax.experimental import pallas as pl
from jax.experimental.pallas import tpu as pltpu
import jax.experimental.pallas.tpu_sc as plsc

_NC = 2
_NS = 16
_NW = _NC * _NS
_CHUNK = 80
_NB = 8
_LAG = 4


def _ring(n_chunks, sl, wl, ss, ws):
    body_groups = (n_chunks - _LAG) // _NB
    tail_start = _LAG + body_groups * _NB

    for b in range(_LAG):
        sl(b, b)
    for i in range(_LAG):
        wl(i, i % _NB)
        ss(i, i % _NB)
        sl(i + _LAG, (i + _LAG) % _NB)

    def step(g, carry):
        i0 = _LAG + g * _NB
        for t in range(_NB):
            i = i0 + t
            b = (_LAG + t) % _NB
            wl(i, b)
            ws((b + _NB - _LAG) % _NB)
            ss(i, b)
            sl(i + _LAG, (b + _NB - _LAG) % _NB)
        return carry

    lax.fori_loop(0, body_groups, step, 0)

    for i in range(tail_start, n_chunks):
        wl(i, i % _NB)
        ws((i + _NB - _LAG) % _NB)
        ss(i, i % _NB)
        if i + _LAG < n_chunks:
            sl(i + _LAG, (i + _NB - _LAG) % _NB)
    for i in range(n_chunks - _LAG, n_chunks):
        ws(i % _NB)


@functools.lru_cache(maxsize=None)
def _make_agg(N, E, D):
    e_per_w = E // _NW
    n_chunks = e_per_w // _CHUNK
    rows_per_tile = -(-N // (_NS * 128)) * 128
    N_pad = rows_per_tile * _NS
    NR = 4
    NI = 8
    mesh = plsc.VectorSubcoreMesh(core_axis_name="c", subcore_axis_name="s")

    def body(y_hbm, src_hbm, dst_hbm, zrow_hbm, out_hbm,
             sidx8, didx8, rows4, acc, *sems):
        semL = sems[:NI]
        semG = sems[NI:NI + NR]
        semS = sems[NI + NR:]
        c = lax.axis_index("c")
        s = lax.axis_index("s")
        wid = c * _NS + s
        r0 = s * rows_per_tile
        base_e = wid * e_per_w

        def sl(i, b):
            pltpu.async_copy(src_hbm.at[pl.ds(base_e + i * _CHUNK, _CHUNK)],
                             sidx8.at[b], semL[b])
            pltpu.async_copy(dst_hbm.at[pl.ds(base_e + i * _CHUNK, _CHUNK)],
                             didx8.at[b], semL[b])

        def wl(i, b):
            pltpu.make_async_copy(
                src_hbm.at[pl.ds(base_e + i * _CHUNK, _CHUNK)],
                sidx8.at[b], semL[b]).wait()
            pltpu.make_async_copy(
                dst_hbm.at[pl.ds(base_e + i * _CHUNK, _CHUNK)],
                didx8.at[b], semL[b]).wait()

        def sg(b, r):
            pltpu.async_copy(y_hbm.at[sidx8.at[b]], rows4.at[r], semG[r])

        def wg(b, r):
            pltpu.make_async_copy(y_hbm.at[sidx8.at[b]], rows4.at[r],
                                  semG[r]).wait()

        def ss(b, r):
            pltpu.async_copy(rows4.at[r], acc.at[didx8.at[b]], semS[r],
                             add=True)

        def ws(b, r):
            pltpu.make_async_copy(rows4.at[r], acc.at[didx8.at[b]],
                                  semS[r]).wait()

        def chunk_body(i, m8, do_next=True, do_ws=True, do_sl=True):
            if do_next:
                wl(i + 2, (m8 + 2) % NI)
            if do_ws:
                ws((m8 - 2) % NI, (m8 - 2) % NR)
            if do_next:
                sg((m8 + 2) % NI, (m8 + 2) % NR)
            wg(m8 % NI, m8 % NR)
            ss(m8 % NI, m8 % NR)
            if do_sl:
                sl(i + 6, (m8 + 6) % NI)

        pltpu.sync_copy(zrow_hbm.at[pl.ds(r0, rows_per_tile)],
                        acc.at[pl.ds(r0, rows_per_tile)])
        plsc.subcore_barrier()

        for i in range(6):
            sl(i, i % NI)
        for i in range(2):
            wl(i, i % NI)
            sg(i % NI, i % NR)

        n = n_chunks
        G = (n - 12) // 8
        for i in range(2):
            chunk_body(i, i % NI, do_ws=False)

        def step(g, carry):
            i0 = 2 + g * 8
            for t in range(8):
                chunk_body(i0 + t, (2 + t) % NI)
            return carry

        lax.fori_loop(0, G, step, 0)
        for i in range(2 + 8 * G, n):
            chunk_body(i, i % NI,
                       do_next=(i + 2 <= n - 1), do_sl=(i + 6 <= n - 1))
        ws((n - 2) % NI, (n - 2) % NR)
        ws((n - 1) % NI, (n - 1) % NR)

        plsc.subcore_barrier()
        pltpu.sync_copy(acc.at[pl.ds(r0, rows_per_tile)],
                        out_hbm.at[pl.ds(c * N_pad + r0, rows_per_tile)])

    return pl.kernel(
        body,
        out_type=jax.ShapeDtypeStruct((_NC * N_pad, D), jnp.float32),
        mesh=mesh,
        scratch_types=(
            pltpu.VMEM((NI, _CHUNK), jnp.int32),
            pltpu.VMEM((NI, _CHUNK), jnp.int32),
            pltpu.VMEM((NR, _CHUNK, D), jnp.float32),
            pltpu.VMEM_SHARED((N_pad, D), jnp.float32),
        ) + (pltpu.SemaphoreType.DMA,) * (NI + 2 * NR),
    )


@functools.lru_cache(maxsize=None)
def _make_count(N, E):
    e_per_w = E // _NW
    n_chunks = e_per_w // _CHUNK
    rows_per_tile = -(-N // (_NS * 128)) * 128
    N_pad = rows_per_tile * _NS
    mesh = plsc.VectorSubcoreMesh(core_axis_name="c", subcore_axis_name="s")

    def body(dst_hbm, zrow_hbm, ones_hbm, out_hbm, didx8, ones_v, acc,
             *sems):
        semL = sems[:_NB]
        semS = sems[_NB:]
        c = lax.axis_index("c")
        s = lax.axis_index("s")
        wid = c * _NS + s
        r0 = s * rows_per_tile
        base_e = wid * e_per_w

        def sl(i, b):
            pltpu.async_copy(dst_hbm.at[pl.ds(base_e + i * _CHUNK, _CHUNK)],
                             didx8.at[b], semL[b])

        def wl(i, b):
            pltpu.make_async_copy(
                dst_hbm.at[pl.ds(base_e + i * _CHUNK, _CHUNK)],
                didx8.at[b], semL[b]).wait()

        def ss(i, b):
            pltpu.async_copy(ones_v, acc.at[didx8.at[b]], semS[b], add=True)

        def ws(b):
            pltpu.make_async_copy(ones_v, acc.at[didx8.at[b]],
                                  semS[b]).wait()

        pltpu.sync_copy(zrow_hbm.at[pl.ds(r0, rows_per_tile)],
                        acc.at[pl.ds(r0, rows_per_tile)])
        pltpu.sync_copy(ones_hbm, ones_v)
        plsc.subcore_barrier()

        _ring(n_chunks, sl, wl, ss, ws)

        plsc.subcore_barrier()
        pltpu.sync_copy(acc.at[pl.ds(r0, rows_per_tile)],
                        out_hbm.at[pl.ds(c * N_pad + r0, rows_per_tile)])

    return pl.kernel(
        body,
        out_type=jax.ShapeDtypeStruct((_NC * N_pad, 128), jnp.float32),
        mesh=mesh,
        scratch_types=(
            pltpu.VMEM((_NB, _CHUNK), jnp.int32),
            pltpu.VMEM((_CHUNK, 128), jnp.float32),
            pltpu.VMEM_SHARED((N_pad, 128), jnp.float32),
        ) + (pltpu.SemaphoreType.DMA,) * (2 * _NB),
    )


@functools.lru_cache(maxsize=None)
def _make_combine(N, D, H, BLOCK):
    grid = N // BLOCK

    def body(p0_ref, p1_ref, c0_ref, c1_ref, x_ref, b_ref, wl_ref, wr_ref,
             o_ref):
        cnt = c0_ref[...] + c1_ref[...]
        inv = 1.0 / jnp.maximum(cnt, 1.0)
        mean = (p0_ref[...] + p1_ref[...]) * inv
        o = (jnp.dot(mean, wl_ref[...], preferred_element_type=jnp.float32)
             + b_ref[...]
             + jnp.dot(x_ref[...], wr_ref[...],
                       preferred_element_type=jnp.float32))
        o_ref[...] = jnp.maximum(o, 0.0)

    row_spec = pl.BlockSpec((BLOCK, D), lambda i: (i, 0))
    cnt_spec = pl.BlockSpec((BLOCK, 1), lambda i: (i, 0))
    full = lambda shape: pl.BlockSpec(shape, lambda i: (0, 0))
    return pl.pallas_call(
        body,
        grid=(grid,),
        in_specs=[row_spec, row_spec, cnt_spec, cnt_spec, row_spec,
                  full((1, H)), full((D, H)), full((D, H))],
        out_specs=pl.BlockSpec((BLOCK, H), lambda i: (i, 0)),
        out_shape=jax.ShapeDtypeStruct((N, H), jnp.float32),
    )


def kernel(x, edge_index, W_l1, b_l1, W_r1, W_l2, b_l2, W_r2):
    N, D = x.shape
    H = W_l1.shape[0]
    O = W_l2.shape[0]
    E = edge_index.shape[1]
    src = edge_index[0]
    dst = edge_index[1]
    rows_per_tile = -(-N // (_NS * 128)) * 128
    N_pad = rows_per_tile * _NS
    zrow = jnp.zeros((N_pad, D), jnp.float32)
    onesrow = jnp.ones((_CHUNK, 128), jnp.float32)

    agg = _make_agg(N, E, D)
    cntk = _make_count(N, E)
    combine1 = _make_combine(N, D, H, 1000)
    combine2 = _make_combine(N, H, O, 1000)

    cout = cntk(dst, zrow, onesrow)
    c0 = cout[:N, 0:1]
    c1 = cout[N_pad:N_pad + N, 0:1]
    x = x + 0.0 * cout[0, 0]
    p = agg(x, src, dst, zrow)
    h = combine1(p[:N], p[N_pad:N_pad + N], c0, c1, x, b_l1.reshape(1, H),
                 W_l1.T, W_r1.T)
    q = agg(h, src, dst, zrow)
    out = combine2(q[:N], q[N_pad:N_pad + N], c0, c1, h, b_l2.reshape(1, O),
                   W_l2.T, W_r2.T)
    return out


# speedup vs baseline: 11.3478x; 1.0639x over previous
import functools

import jax
import jax.numpy as jnp
from jax import lax
from jax.experimental import pallas as pl
from jax.experimental.pallas import tpu as pltpu
import jax.experimental.pallas.tpu_sc as plsc

_NC = 2
_NS = 16
_NW = _NC * _NS
_CHUNK = 80
_NB = 8
_LAG = 4


def _ring(n_chunks, sl, wl, ss, ws):
    body_groups = (n_chunks - _LAG) // _NB
    tail_start = _LAG + body_groups * _NB

    for b in range(_LAG):
        sl(b, b)
    for i in range(_LAG):
        wl(i, i % _NB)
        ss(i, i % _NB)
        sl(i + _LAG, (i + _LAG) % _NB)

    def step(g, carry):
        i0 = _LAG + g * _NB
        for t in range(_NB):
            i = i0 + t
            b = (_LAG + t) % _NB
            wl(i, b)
            ws((b + _NB - _LAG) % _NB)
            ss(i, b)
            sl(i + _LAG, (b + _NB - _LAG) % _NB)
        return carry

    lax.fori_loop(0, body_groups, step, 0)

    for i in range(tail_start, n_chunks):
        wl(i, i % _NB)
        ws((i + _NB - _LAG) % _NB)
        ss(i, i % _NB)
        if i + _LAG < n_chunks:
            sl(i + _LAG, (i + _NB - _LAG) % _NB)
    for i in range(n_chunks - _LAG, n_chunks):
        ws(i % _NB)


@functools.lru_cache(maxsize=None)
def _make_agg(N, E, D):
    e_per_w = E // _NW
    n_chunks = e_per_w // _CHUNK
    rows_per_tile = -(-N // (_NS * 128)) * 128
    N_pad = rows_per_tile * _NS
    NR = 4
    NI = 8
    mesh = plsc.VectorSubcoreMesh(core_axis_name="c", subcore_axis_name="s")

    def body(y_hbm, src_hbm, dst_hbm, zrow_hbm, out_hbm,
             sidx8, didx8, rows4, acc, *sems):
        semL = sems[:NI]
        semG = sems[NI:NI + NR]
        semS = sems[NI + NR:]
        c = lax.axis_index("c")
        s = lax.axis_index("s")
        wid = c * _NS + s
        r0 = s * rows_per_tile
        base_e = wid * e_per_w

        def sl(i, b):
            pltpu.async_copy(src_hbm.at[pl.ds(base_e + i * _CHUNK, _CHUNK)],
                             sidx8.at[b], semL[b])
            pltpu.async_copy(dst_hbm.at[pl.ds(base_e + i * _CHUNK, _CHUNK)],
                             didx8.at[b], semL[b])

        def wl(i, b):
            pltpu.make_async_copy(
                src_hbm.at[pl.ds(base_e + i * _CHUNK, _CHUNK)],
                sidx8.at[b], semL[b]).wait()
            pltpu.make_async_copy(
                dst_hbm.at[pl.ds(base_e + i * _CHUNK, _CHUNK)],
                didx8.at[b], semL[b]).wait()

        def sg(b, r):
            pltpu.async_copy(y_hbm.at[sidx8.at[b]], rows4.at[r], semG[r])

        def wg(b, r):
            pltpu.make_async_copy(y_hbm.at[sidx8.at[b]], rows4.at[r],
                                  semG[r]).wait()

        def ss(b, r):
            pltpu.async_copy(rows4.at[r], acc.at[didx8.at[b]], semS[r],
                             add=True)

        def ws(b, r):
            pltpu.make_async_copy(rows4.at[r], acc.at[didx8.at[b]],
                                  semS[r]).wait()

        def chunk_body(i, m8, do_next=True, do_ws=True, do_sl=True):
            if do_next:
                wl(i + 2, (m8 + 2) % NI)
            if do_ws:
                ws((m8 - 2) % NI, (m8 - 2) % NR)
            if do_next:
                sg((m8 + 2) % NI, (m8 + 2) % NR)
            wg(m8 % NI, m8 % NR)
            ss(m8 % NI, m8 % NR)
            if do_sl:
                sl(i + 6, (m8 + 6) % NI)

        pltpu.sync_copy(zrow_hbm.at[pl.ds(r0, rows_per_tile)],
                        acc.at[pl.ds(r0, rows_per_tile)])
        plsc.subcore_barrier()

        for i in range(6):
            sl(i, i % NI)
        for i in range(2):
            wl(i, i % NI)
            sg(i % NI, i % NR)

        n = n_chunks
        G = (n - 12) // 8
        for i in range(2):
            chunk_body(i, i % NI, do_ws=False)

        def step(g, carry):
            i0 = 2 + g * 8
            for t in range(8):
                chunk_body(i0 + t, (2 + t) % NI)
            return carry

        lax.fori_loop(0, G, step, 0)
        for i in range(2 + 8 * G, n):
            chunk_body(i, i % NI,
                       do_next=(i + 2 <= n - 1), do_sl=(i + 6 <= n - 1))
        ws((n - 2) % NI, (n - 2) % NR)
        ws((n - 1) % NI, (n - 1) % NR)

        plsc.subcore_barrier()
        pltpu.sync_copy(acc.at[pl.ds(r0, rows_per_tile)],
                        out_hbm.at[pl.ds(c * N_pad + r0, rows_per_tile)])

    return pl.kernel(
        body,
        out_type=jax.ShapeDtypeStruct((_NC * N_pad, D), jnp.float32),
        mesh=mesh,
        scratch_types=(
            pltpu.VMEM((NI, _CHUNK), jnp.int32),
            pltpu.VMEM((NI, _CHUNK), jnp.int32),
            pltpu.VMEM((NR, _CHUNK, D), jnp.float32),
            pltpu.VMEM_SHARED((N_pad, D), jnp.float32),
        ) + (pltpu.SemaphoreType.DMA,) * (NI + 2 * NR),
    )


@functools.lru_cache(maxsize=None)
def _make_count(N, E):
    e_per_w = E // _NW
    n_chunks = e_per_w // _CHUNK
    rows_per_tile = -(-N // (_NS * 128)) * 128
    N_pad = rows_per_tile * _NS
    mesh = plsc.VectorSubcoreMesh(core_axis_name="c", subcore_axis_name="s")

    def body(dst_hbm, zrow_hbm, ones_hbm, out_hbm, didx8, ones_v, acc,
             *sems):
        semL = sems[:_NB]
        semS = sems[_NB:]
        c = lax.axis_index("c")
        s = lax.axis_index("s")
        wid = c * _NS + s
        r0 = s * rows_per_tile
        base_e = wid * e_per_w

        def sl(i, b):
            pltpu.async_copy(dst_hbm.at[pl.ds(base_e + i * _CHUNK, _CHUNK)],
                             didx8.at[b], semL[b])

        def wl(i, b):
            pltpu.make_async_copy(
                dst_hbm.at[pl.ds(base_e + i * _CHUNK, _CHUNK)],
                didx8.at[b], semL[b]).wait()

        def ss(i, b):
            pltpu.async_copy(ones_v, acc.at[didx8.at[b]], semS[b], add=True)

        def ws(b):
            pltpu.make_async_copy(ones_v, acc.at[didx8.at[b]],
                                  semS[b]).wait()

        pltpu.sync_copy(zrow_hbm.at[pl.ds(r0, rows_per_tile)],
                        acc.at[pl.ds(r0, rows_per_tile)])
        pltpu.sync_copy(ones_hbm, ones_v)
        plsc.subcore_barrier()

        _ring(n_chunks, sl, wl, ss, ws)

        plsc.subcore_barrier()
        pltpu.sync_copy(acc.at[pl.ds(r0, rows_per_tile)],
                        out_hbm.at[pl.ds(c * N_pad + r0, rows_per_tile)])

    return pl.kernel(
        body,
        out_type=jax.ShapeDtypeStruct((_NC * N_pad, 128), jnp.float32),
        mesh=mesh,
        scratch_types=(
            pltpu.VMEM((_NB, _CHUNK), jnp.int32),
            pltpu.VMEM((_CHUNK, 128), jnp.float32),
            pltpu.VMEM_SHARED((N_pad, 128), jnp.float32),
        ) + (pltpu.SemaphoreType.DMA,) * (2 * _NB),
    )


@functools.lru_cache(maxsize=None)
def _make_combine(N, D, H, BLOCK):
    grid = N // BLOCK

    def body(p0_ref, p1_ref, c0_ref, c1_ref, x_ref, b_ref, wl_ref, wr_ref,
             o_ref):
        cnt = c0_ref[0] + c1_ref[0]
        inv = 1.0 / jnp.maximum(cnt, 1.0)
        mean = (p0_ref[0] + p1_ref[0]) * inv
        o = (jnp.dot(mean, wl_ref[...], preferred_element_type=jnp.float32)
             + b_ref[...]
             + jnp.dot(x_ref[...], wr_ref[...],
                       preferred_element_type=jnp.float32))
        o_ref[...] = jnp.maximum(o, 0.0)

    p_spec = lambda c: pl.BlockSpec((1, BLOCK, D), lambda i, c=c: (c, i, 0))
    c_spec = lambda c: pl.BlockSpec((1, BLOCK, 1), lambda i, c=c: (c, i, 0))
    row_spec = pl.BlockSpec((BLOCK, D), lambda i: (i, 0))
    full = lambda shape: pl.BlockSpec(shape, lambda i: (0, 0))
    return pl.pallas_call(
        body,
        grid=(grid,),
        in_specs=[p_spec(0), p_spec(1), c_spec(0), c_spec(1), row_spec,
                  full((1, H)), full((D, H)), full((D, H))],
        out_specs=pl.BlockSpec((BLOCK, H), lambda i: (i, 0)),
        out_shape=jax.ShapeDtypeStruct((N, H), jnp.float32),
    )


def kernel(x, edge_index, W_l1, b_l1, W_r1, W_l2, b_l2, W_r2):
    N, D = x.shape
    H = W_l1.shape[0]
    O = W_l2.shape[0]
    E = edge_index.shape[1]
    src = edge_index[0]
    dst = edge_index[1]
    rows_per_tile = -(-N // (_NS * 128)) * 128
    N_pad = rows_per_tile * _NS
    zrow = jnp.zeros((N_pad, D), jnp.float32)
    onesrow = jnp.ones((_CHUNK, 128), jnp.float32)

    agg = _make_agg(N, E, D)
    cntk = _make_count(N, E)
    combine1 = _make_combine(N, D, H, 1000)
    combine2 = _make_combine(N, H, O, 1000)

    cout = cntk(dst, zrow, onesrow).reshape(_NC, N_pad, 128)[:, :, 0:1]
    p = agg(x, src, dst, zrow).reshape(_NC, N_pad, D)
    h = combine1(p, p, cout, cout, x, b_l1.reshape(1, H),
                 W_l1.T, W_r1.T)
    q = agg(h, src, dst, zrow).reshape(_NC, N_pad, H)
    out = combine2(q, q, cout, cout, h, b_l2.reshape(1, O),
                   W_l2.T, W_r2.T)
    return out


# speedup vs baseline: 11.4441x; 1.0085x over previous
import functools

import jax
import jax.numpy as jnp
from jax import lax
from jax.experimental import pallas as pl
from jax.experimental.pallas import tpu as pltpu
import jax.experimental.pallas.tpu_sc as plsc

_NC = 2
_NS = 16
_NW = _NC * _NS
_CHUNK = 80
_NB = 8
_LAG = 4


def _ring(n_chunks, sl, wl, ss, ws, primed=False):
    body_groups = (n_chunks - _LAG) // _NB
    tail_start = _LAG + body_groups * _NB

    if not primed:
        for b in range(_LAG):
            sl(b, b)
    for i in range(_LAG):
        wl(i, i % _NB)
        ss(i, i % _NB)
        sl(i + _LAG, (i + _LAG) % _NB)

    def step(g, carry):
        i0 = _LAG + g * _NB
        for t in range(_NB):
            i = i0 + t
            b = (_LAG + t) % _NB
            wl(i, b)
            ws((b + _NB - _LAG) % _NB)
            ss(i, b)
            sl(i + _LAG, (b + _NB - _LAG) % _NB)
        return carry

    lax.fori_loop(0, body_groups, step, 0)

    for i in range(tail_start, n_chunks):
        wl(i, i % _NB)
        ws((i + _NB - _LAG) % _NB)
        ss(i, i % _NB)
        if i + _LAG < n_chunks:
            sl(i + _LAG, (i + _NB - _LAG) % _NB)
    for i in range(n_chunks - _LAG, n_chunks):
        ws(i % _NB)


@functools.lru_cache(maxsize=None)
def _make_agg(N, E, D):
    e_per_w = E // _NW
    n_chunks = e_per_w // _CHUNK
    rows_per_tile = -(-N // (_NS * 128)) * 128
    N_pad = rows_per_tile * _NS
    NR = 4
    NI = 8
    mesh = plsc.VectorSubcoreMesh(core_axis_name="c", subcore_axis_name="s")

    def body(y_hbm, src_hbm, dst_hbm, zrow_hbm, out_hbm,
             sidx8, didx8, rows4, acc, *sems):
        semL = sems[:NI]
        semG = sems[NI:NI + NR]
        semS = sems[NI + NR:]
        c = lax.axis_index("c")
        s = lax.axis_index("s")
        wid = c * _NS + s
        r0 = s * rows_per_tile
        base_e = wid * e_per_w

        def sl(i, b):
            pltpu.async_copy(src_hbm.at[pl.ds(base_e + i * _CHUNK, _CHUNK)],
                             sidx8.at[b], semL[b])
            pltpu.async_copy(dst_hbm.at[pl.ds(base_e + i * _CHUNK, _CHUNK)],
                             didx8.at[b], semL[b])

        def wl(i, b):
            pltpu.make_async_copy(
                src_hbm.at[pl.ds(base_e + i * _CHUNK, _CHUNK)],
                sidx8.at[b], semL[b]).wait()
            pltpu.make_async_copy(
                dst_hbm.at[pl.ds(base_e + i * _CHUNK, _CHUNK)],
                didx8.at[b], semL[b]).wait()

        def sg(b, r):
            pltpu.async_copy(y_hbm.at[sidx8.at[b]], rows4.at[r], semG[r])

        def wg(b, r):
            pltpu.make_async_copy(y_hbm.at[sidx8.at[b]], rows4.at[r],
                                  semG[r]).wait()

        def ss(b, r):
            pltpu.async_copy(rows4.at[r], acc.at[didx8.at[b]], semS[r],
                             add=True)

        def ws(b, r):
            pltpu.make_async_copy(rows4.at[r], acc.at[didx8.at[b]],
                                  semS[r]).wait()

        def chunk_body(i, m8, do_next=True, do_ws=True, do_sl=True):
            if do_next:
                wl(i + 2, (m8 + 2) % NI)
            if do_ws:
                ws((m8 - 2) % NI, (m8 - 2) % NR)
            if do_next:
                sg((m8 + 2) % NI, (m8 + 2) % NR)
            wg(m8 % NI, m8 % NR)
            ss(m8 % NI, m8 % NR)
            if do_sl:
                sl(i + 6, (m8 + 6) % NI)

        for i in range(6):
            sl(i, i % NI)
        pltpu.sync_copy(zrow_hbm.at[pl.ds(r0, rows_per_tile)],
                        acc.at[pl.ds(r0, rows_per_tile)])
        for i in range(2):
            wl(i, i % NI)
            sg(i % NI, i % NR)
        plsc.subcore_barrier()

        n = n_chunks
        G = (n - 12) // 8
        for i in range(2):
            chunk_body(i, i % NI, do_ws=False)

        def step(g, carry):
            i0 = 2 + g * 8
            for t in range(8):
                chunk_body(i0 + t, (2 + t) % NI)
            return carry

        lax.fori_loop(0, G, step, 0)
        for i in range(2 + 8 * G, n):
            chunk_body(i, i % NI,
                       do_next=(i + 2 <= n - 1), do_sl=(i + 6 <= n - 1))
        ws((n - 2) % NI, (n - 2) % NR)
        ws((n - 1) % NI, (n - 1) % NR)

        plsc.subcore_barrier()
        pltpu.sync_copy(acc.at[pl.ds(r0, rows_per_tile)],
                        out_hbm.at[pl.ds(c * N_pad + r0, rows_per_tile)])

    return pl.kernel(
        body,
        out_type=jax.ShapeDtypeStruct((_NC * N_pad, D), jnp.float32),
        mesh=mesh,
        scratch_types=(
            pltpu.VMEM((NI, _CHUNK), jnp.int32),
            pltpu.VMEM((NI, _CHUNK), jnp.int32),
            pltpu.VMEM((NR, _CHUNK, D), jnp.float32),
            pltpu.VMEM_SHARED((N_pad, D), jnp.float32),
        ) + (pltpu.SemaphoreType.DMA,) * (NI + 2 * NR),
    )


@functools.lru_cache(maxsize=None)
def _make_count(N, E):
    e_per_w = E // _NW
    n_chunks = e_per_w // _CHUNK
    rows_per_tile = -(-N // (_NS * 128)) * 128
    N_pad = rows_per_tile * _NS
    mesh = plsc.VectorSubcoreMesh(core_axis_name="c", subcore_axis_name="s")

    def body(dst_hbm, zrow_hbm, ones_hbm, out_hbm, didx8, ones_v, acc,
             *sems):
        semL = sems[:_NB]
        semS = sems[_NB:]
        c = lax.axis_index("c")
        s = lax.axis_index("s")
        wid = c * _NS + s
        r0 = s * rows_per_tile
        base_e = wid * e_per_w

        def sl(i, b):
            pltpu.async_copy(dst_hbm.at[pl.ds(base_e + i * _CHUNK, _CHUNK)],
                             didx8.at[b], semL[b])

        def wl(i, b):
            pltpu.make_async_copy(
                dst_hbm.at[pl.ds(base_e + i * _CHUNK, _CHUNK)],
                didx8.at[b], semL[b]).wait()

        def ss(i, b):
            pltpu.async_copy(ones_v, acc.at[didx8.at[b]], semS[b], add=True)

        def ws(b):
            pltpu.make_async_copy(ones_v, acc.at[didx8.at[b]],
                                  semS[b]).wait()

        for b in range(_LAG):
            sl(b, b)
        pltpu.sync_copy(zrow_hbm.at[pl.ds(r0, rows_per_tile)],
                        acc.at[pl.ds(r0, rows_per_tile)])
        pltpu.sync_copy(ones_hbm, ones_v)
        plsc.subcore_barrier()

        _ring(n_chunks, sl, wl, ss, ws, primed=True)

        plsc.subcore_barrier()
        pltpu.sync_copy(acc.at[pl.ds(r0, rows_per_tile)],
                        out_hbm.at[pl.ds(c * N_pad + r0, rows_per_tile)])

    return pl.kernel(
        body,
        out_type=jax.ShapeDtypeStruct((_NC * N_pad, 128), jnp.float32),
        mesh=mesh,
        scratch_types=(
            pltpu.VMEM((_NB, _CHUNK), jnp.int32),
            pltpu.VMEM((_CHUNK, 128), jnp.float32),
            pltpu.VMEM_SHARED((N_pad, 128), jnp.float32),
        ) + (pltpu.SemaphoreType.DMA,) * (2 * _NB),
    )


@functools.lru_cache(maxsize=None)
def _make_combine(N, D, H, BLOCK):
    grid = N // BLOCK

    def body(p0_ref, p1_ref, c0_ref, c1_ref, x_ref, b_ref, wl_ref, wr_ref,
             o_ref):
        cnt = c0_ref[0] + c1_ref[0]
        inv = 1.0 / jnp.maximum(cnt, 1.0)
        mean = (p0_ref[0] + p1_ref[0]) * inv
        o = (jnp.dot(mean, wl_ref[...], preferred_element_type=jnp.float32)
             + b_ref[...]
             + jnp.dot(x_ref[...], wr_ref[...],
                       preferred_element_type=jnp.float32))
        o_ref[...] = jnp.maximum(o, 0.0)

    p_spec = lambda c: pl.BlockSpec((1, BLOCK, D), lambda i, c=c: (c, i, 0))
    c_spec = lambda c: pl.BlockSpec((1, BLOCK, 1), lambda i, c=c: (c, i, 0))
    row_spec = pl.BlockSpec((BLOCK, D), lambda i: (i, 0))
    full = lambda shape: pl.BlockSpec(shape, lambda i: (0, 0))
    return pl.pallas_call(
        body,
        grid=(grid,),
        in_specs=[p_spec(0), p_spec(1), c_spec(0), c_spec(1), row_spec,
                  full((1, H)), full((D, H)), full((D, H))],
        out_specs=pl.BlockSpec((BLOCK, H), lambda i: (i, 0)),
        out_shape=jax.ShapeDtypeStruct((N, H), jnp.float32),
    )


def kernel(x, edge_index, W_l1, b_l1, W_r1, W_l2, b_l2, W_r2):
    N, D = x.shape
    H = W_l1.shape[0]
    O = W_l2.shape[0]
    E = edge_index.shape[1]
    src = edge_index[0]
    dst = edge_index[1]
    rows_per_tile = -(-N // (_NS * 128)) * 128
    N_pad = rows_per_tile * _NS
    zrow = jnp.zeros((N_pad, D), jnp.float32)
    onesrow = jnp.ones((_CHUNK, 128), jnp.float32)

    agg = _make_agg(N, E, D)
    cntk = _make_count(N, E)
    combine1 = _make_combine(N, D, H, 1000)
    combine2 = _make_combine(N, H, O, 1000)

    cout = cntk(dst, zrow, onesrow).reshape(_NC, N_pad, 128)[:, :, 0:1]
    p = agg(x, src, dst, zrow).reshape(_NC, N_pad, D)
    h = combine1(p, p, cout, cout, x, b_l1.reshape(1, H),
                 W_l1.T, W_r1.T)
    q = agg(h, src, dst, zrow).reshape(_NC, N_pad, H)
    out = combine2(q, q, cout, cout, h, b_l2.reshape(1, O),
                   W_l2.T, W_r2.T)
    return out
